# baseline (speedup 1.0000x reference)
import math

import jax
import jax.numpy as jnp
from jax.experimental import pallas as pl
from jax.experimental.pallas import tpu as pltpu

BSZ = 8
C_LEN = 512
QN = 64
QL = 64
D = 512
H = 512
G = 8


def _question(cp, qp_b, cat):
    st = jax.lax.dot_general(
        qp_b, cp, (((1,), (1,)), ((), ())),
        preferred_element_type=jnp.float32,
    ) * (1.0 / math.sqrt(H))
    st = st - jnp.max(st, axis=0, keepdims=True)
    e = jnp.exp(st)
    attn_t = (e / jnp.sum(e, axis=0, keepdims=True)).astype(jnp.bfloat16)
    return jax.lax.dot_general(
        attn_t, cat, (((0,), (0,)), ((), ())),
        preferred_element_type=jnp.float32)


def _fused_kernel(tags_ref, ctx_ref, q_ref, w1_ref, w2_ref,
                  out_ref, cp_all, qsum_scr, pos_ref):
    k = pl.program_id(0)

    @pl.when(k == 0)
    def _():
        for b in range(BSZ):
            cp_all[b] = jax.nn.relu(
                jnp.dot(ctx_ref[b], w1_ref[...], preferred_element_type=jnp.float32)
            ).astype(jnp.bfloat16)

    p = pos_ref[0]
    S = qsum_scr[...]
    tprev = tags_ref[jnp.maximum(G * k - 1, 0)]
    tis = []
    cats = []
    for i in range(G):
        ti = tags_ref[G * k + i]
        seg = (ti != tprev) if i else ((k == 0) | (ti != tprev))
        p = jnp.where(seg, 0, p)
        S = jnp.where(seg, 0.0, S)
        inv = jnp.where(p == 0, 0.0, 1.0 / p.astype(jnp.float32))
        avg_b = (S * inv).astype(jnp.bfloat16)
        qf = q_ref[0, i]
        cats.append(jnp.concatenate(
            [qf.astype(jnp.bfloat16), avg_b], axis=1))
        tis.append(ti)
        S = S + qf
        p = p + 1
        tprev = ti
    qsum_scr[...] = S
    pos_ref[0] = p

    cat4 = jnp.concatenate(cats, axis=0)
    qp4 = jnp.dot(cat4, w2_ref[...], preferred_element_type=jnp.float32)
    qp4_b = jax.nn.relu(qp4).astype(jnp.bfloat16)

    for i in range(G):
        out_ref[0, i] = _question(cp_all[tis[i]],
                                  qp4_b[i * QL:(i + 1) * QL], cats[i])


def kernel(contexts, questions, tags, W1, W2):
    tags32 = tags.astype(jnp.int32)
    ctx_b = contexts.astype(jnp.bfloat16)
    w1_b = W1.astype(jnp.bfloat16)
    w2_b = W2.astype(jnp.bfloat16)
    q4 = questions.reshape(QN // G, G, QL, D)

    out = pl.pallas_call(
        _fused_kernel,
        grid_spec=pltpu.PrefetchScalarGridSpec(
            num_scalar_prefetch=1,
            grid=(QN // G,),
            in_specs=[
                pl.BlockSpec((BSZ, C_LEN, D), lambda k, t: (0, 0, 0)),
                pl.BlockSpec((1, G, QL, D), lambda k, t: (k, 0, 0, 0)),
                pl.BlockSpec((D, H), lambda k, t: (0, 0)),
                pl.BlockSpec((2 * D, H), lambda k, t: (0, 0)),
            ],
            out_specs=pl.BlockSpec((1, G, C_LEN, 2 * D), lambda k, t: (k, 0, 0, 0)),
            scratch_shapes=[
                pltpu.VMEM((BSZ, C_LEN, H), jnp.bfloat16),
                pltpu.VMEM((QL, D), jnp.float32),
                pltpu.SMEM((1,), jnp.int32),
            ],
        ),
        out_shape=jax.ShapeDtypeStruct((QN // G, G, C_LEN, 2 * D), jnp.float32),
        compiler_params=pltpu.CompilerParams(dimension_semantics=("arbitrary",)),
    )(tags32, ctx_b, q4, w1_b, w2_b)

    return out.reshape(QN, C_LEN, 2 * D)


# speedup vs baseline: 1.1447x; 1.1447x over previous
import math

import jax
import jax.numpy as jnp
from jax.experimental import pallas as pl
from jax.experimental.pallas import tpu as pltpu

BSZ = 8
C_LEN = 512
QN = 64
QL = 64
D = 512
H = 512
G = 8


def _question(cp, qp_b, cat):
    st = jax.lax.dot_general(
        qp_b, cp, (((1,), (1,)), ((), ())),
        preferred_element_type=jnp.float32,
    ) * (1.0 / math.sqrt(H))
    st = st - jnp.max(st, axis=0, keepdims=True)
    e = jnp.exp(st)
    attn_t = (e / jnp.sum(e, axis=0, keepdims=True)).astype(jnp.bfloat16)
    return jax.lax.dot_general(
        attn_t, cat, (((0,), (0,)), ((), ())),
        preferred_element_type=jnp.float32)


def _fused_kernel(tags_ref, ctx_ref, q_ref, w1_ref, w2_ref,
                  out_ref, cp_all, w2_scr, qsum_scr, pos_ref):
    k = pl.program_id(0)

    @pl.when(k == 0)
    def _():
        w2_scr[...] = w2_ref[...].astype(jnp.bfloat16)
        w1b = w1_ref[...].astype(jnp.bfloat16)
        for b in range(BSZ):
            cp_all[b] = jax.nn.relu(
                jnp.dot(ctx_ref[b].astype(jnp.bfloat16), w1b,
                        preferred_element_type=jnp.float32)
            ).astype(jnp.bfloat16)

    p = pos_ref[0]
    S = qsum_scr[...]
    tprev = tags_ref[jnp.maximum(G * k - 1, 0)]
    tis = []
    cats = []
    for i in range(G):
        ti = tags_ref[G * k + i]
        seg = (ti != tprev) if i else ((k == 0) | (ti != tprev))
        p = jnp.where(seg, 0, p)
        S = jnp.where(seg, 0.0, S)
        inv = jnp.where(p == 0, 0.0, 1.0 / p.astype(jnp.float32))
        avg_b = (S * inv).astype(jnp.bfloat16)
        qf = q_ref[0, i]
        cats.append(jnp.concatenate(
            [qf.astype(jnp.bfloat16), avg_b], axis=1))
        tis.append(ti)
        S = S + qf
        p = p + 1
        tprev = ti
    qsum_scr[...] = S
    pos_ref[0] = p

    cat4 = jnp.concatenate(cats, axis=0)
    qp4 = jnp.dot(cat4, w2_scr[...], preferred_element_type=jnp.float32)
    qp4_b = jax.nn.relu(qp4).astype(jnp.bfloat16)

    for i in range(G):
        out_ref[0, i] = _question(cp_all[tis[i]],
                                  qp4_b[i * QL:(i + 1) * QL], cats[i])


def kernel(contexts, questions, tags, W1, W2):
    tags32 = tags.astype(jnp.int32)
    q4 = questions.reshape(QN // G, G, QL, D)

    out = pl.pallas_call(
        _fused_kernel,
        grid_spec=pltpu.PrefetchScalarGridSpec(
            num_scalar_prefetch=1,
            grid=(QN // G,),
            in_specs=[
                pl.BlockSpec((BSZ, C_LEN, D), lambda k, t: (0, 0, 0)),
                pl.BlockSpec((1, G, QL, D), lambda k, t: (k, 0, 0, 0)),
                pl.BlockSpec((D, H), lambda k, t: (0, 0)),
                pl.BlockSpec((2 * D, H), lambda k, t: (0, 0)),
            ],
            out_specs=pl.BlockSpec((1, G, C_LEN, 2 * D), lambda k, t: (k, 0, 0, 0)),
            scratch_shapes=[
                pltpu.VMEM((BSZ, C_LEN, H), jnp.bfloat16),
                pltpu.VMEM((2 * D, H), jnp.bfloat16),
                pltpu.VMEM((QL, D), jnp.float32),
                pltpu.SMEM((1,), jnp.int32),
            ],
        ),
        out_shape=jax.ShapeDtypeStruct((QN // G, G, C_LEN, 2 * D), jnp.float32),
        compiler_params=pltpu.CompilerParams(dimension_semantics=("arbitrary",)),
    )(tags32, contexts, q4, W1, W2)

    return out.reshape(QN, C_LEN, 2 * D)
